# baseline (speedup 1.0000x reference)
import jax
import jax.numpy as jnp
from jax.experimental import pallas as pl
from jax.experimental.pallas import tpu as pltpu

DIM_ = 768
EMB_ = 512
N_ = 4096
TOPK_ = 9
CLSP_ = 1024
TQ_ = 256
CB_ = CLSP_ // TQ_
NT_ = 4 * CB_
SCALE_K = 14.0 ** 0.5


def _body(x_hbm, q_ref, wk_ref, wv_ref, wp_ref, bp_ref, o_ref,
          x_s, k_s, v_s, sa_s, sb_s, m_s, sem):
    t = pl.program_id(0)

    def scores_into(dst):
        s = jax.lax.dot_general(q_ref[...], k_s[...], (((1,), (1,)), ((), ())),
                                preferred_element_type=jnp.float32)
        dst[...] = s * jnp.float32(SCALE_K)

    def select_av(prev, cur):
        scores_into(cur)

        s0 = prev[...]
        iota = jax.lax.broadcasted_iota(jnp.int32, (TQ_, N_), 1)
        neginf = jnp.float32(-jnp.inf)

        mx0 = jnp.max(s0, axis=1, keepdims=True)
        am0 = jnp.min(jnp.where(s0 == mx0, iota, N_), axis=1, keepdims=True)
        m_s[...] = jnp.where(iota == am0, neginf, s0)

        def step(i, carry):
            sw = m_s[...]
            mx = jnp.max(sw, axis=1, keepdims=True)
            am = jnp.min(jnp.where(sw == mx, iota, N_), axis=1, keepdims=True)
            m_s[...] = jnp.where(iota == am, neginf, sw)
            return carry

        jax.lax.fori_loop(0, TOPK_ - 1, step, 0, unroll=TOPK_ - 1)

        mask = m_s[...] == neginf
        vals = jnp.where(mask, s0, 0.0)
        nrm = jnp.sqrt(jnp.sum(vals * vals, axis=1, keepdims=True))
        c2 = jnp.float32(N_) / nrm
        mrow = jnp.maximum(mx0 * c2, 0.0)
        e = jnp.exp(jnp.where(mask, s0 * c2, 0.0) - mrow)
        z = jnp.sum(e, axis=1, keepdims=True)
        w = e / z

        o1 = jnp.dot(w, v_s[...], preferred_element_type=jnp.float32)
        o = jnp.dot(o1, wp_ref[...], preferred_element_type=jnp.float32)
        o_ref[0] = o + bp_ref[...]

    @pl.when(t % 2 == 0)
    def _even():
        select_av(sb_s, sa_s)

    @pl.when(t % 2 == 1)
    def _odd():
        select_av(sa_s, sb_s)

    boundary = jnp.logical_and(t % CB_ == 0, t < NT_)

    @pl.when(boundary)
    def _proj_kv():
        for c in range(4):
            sl = pl.ds(c * (N_ // 4), N_ // 4)
            cp = pltpu.make_async_copy(x_hbm.at[t // CB_, sl], x_s, sem)
            cp.start()
            cp.wait()
            xc = x_s[...]
            k_s[sl] = jnp.dot(xc, wk_ref[...], preferred_element_type=jnp.float32)
            v_s[sl] = jnp.dot(xc, wv_ref[...], preferred_element_type=jnp.float32)

    @pl.when(jnp.logical_and(boundary, t % 2 == 0))
    def _rescore_a():
        scores_into(sa_s)

    @pl.when(jnp.logical_and(boundary, t % 2 == 1))
    def _rescore_b():
        scores_into(sb_s)


def _prev(t):
    tp = jnp.maximum(t - 1, 0)
    return tp // CB_, tp % CB_


@jax.jit
def kernel(x, q, Wk, Wv, Wp, bp):
    B, N, C = x.shape
    CLS = q.shape[0]
    qp = jnp.pad(q, ((0, CLSP_ - CLS), (0, 0)))
    bp2 = bp.reshape(1, DIM_)
    out = pl.pallas_call(
        _body,
        grid=(NT_ + 1,),
        in_specs=[
            pl.BlockSpec(memory_space=pl.ANY),
            pl.BlockSpec((TQ_, EMB_),
                         lambda t: (jnp.minimum(t, NT_ - 1) % CB_, 0)),
            pl.BlockSpec((DIM_, EMB_), lambda t: (0, 0)),
            pl.BlockSpec((DIM_, EMB_), lambda t: (0, 0)),
            pl.BlockSpec((EMB_, DIM_), lambda t: (0, 0)),
            pl.BlockSpec((1, DIM_), lambda t: (0, 0)),
        ],
        out_specs=pl.BlockSpec(
            (1, TQ_, DIM_), lambda t: (*_prev(t), 0)),
        out_shape=jax.ShapeDtypeStruct((4, CLSP_, DIM_), jnp.float32),
        scratch_shapes=[
            pltpu.VMEM((N_ // 4, DIM_), jnp.float32),
            pltpu.VMEM((N_, EMB_), jnp.float32),
            pltpu.VMEM((N_, EMB_), jnp.float32),
            pltpu.VMEM((TQ_, N_), jnp.float32),
            pltpu.VMEM((TQ_, N_), jnp.float32),
            pltpu.VMEM((TQ_, N_), jnp.float32),
            pltpu.SemaphoreType.DMA,
        ],
    )(x, qp, Wk, Wv, Wp, bp2)
    return out[:, :CLS, :]


# speedup vs baseline: 2.2062x; 2.2062x over previous
import jax
import jax.numpy as jnp
from jax.experimental import pallas as pl
from jax.experimental.pallas import tpu as pltpu

DIM_ = 768
EMB_ = 512
N_ = 4096
TOPK_ = 9
CLSP_ = 1024
TQ_ = 256
SCALE_K = 14.0 ** 0.5


def _body(x_hbm, q_ref, wk_ref, wv_ref, wp_ref, bp_ref, o_ref,
          x_s, k_s, v_s, s_s, m_s, sem):
    b = pl.program_id(0)
    j = pl.program_id(1)

    @pl.when(j == 0)
    def _proj_kv():
        cp = pltpu.make_async_copy(x_hbm.at[b], x_s, sem)
        cp.start()
        cp.wait()
        xb = x_s[...]
        k_s[...] = jnp.dot(xb, wk_ref[...], preferred_element_type=jnp.float32)
        v_s[...] = jnp.dot(xb, wv_ref[...], preferred_element_type=jnp.float32)

    s = jax.lax.dot_general(q_ref[...], k_s[...], (((1,), (1,)), ((), ())),
                            preferred_element_type=jnp.float32)
    s = s * jnp.float32(SCALE_K)
    s_s[...] = s
    m_s[...] = s

    iota = jax.lax.broadcasted_iota(jnp.int32, (TQ_, N_), 1)
    neginf = jnp.float32(-jnp.inf)

    def step(t, carry):
        sw = m_s[...]
        mx = jnp.max(sw, axis=1, keepdims=True)
        am = jnp.min(jnp.where(sw == mx, iota, N_), axis=1, keepdims=True)
        m_s[...] = jnp.where(iota == am, neginf, sw)
        return carry

    jax.lax.fori_loop(0, TOPK_, step, 0)

    s0 = s_s[...]
    mask = m_s[...] == neginf
    vals = jnp.where(mask, s0, 0.0)
    nrm = jnp.sqrt(jnp.sum(vals * vals, axis=1, keepdims=True))
    c2 = jnp.float32(N_) / nrm
    mrow = jnp.maximum(jnp.max(s0, axis=1, keepdims=True) * c2, 0.0)
    e = jnp.exp(jnp.where(mask, s0 * c2, 0.0) - mrow)
    z = jnp.sum(e, axis=1, keepdims=True)
    w = e / z

    o1 = jnp.dot(w, v_s[...], preferred_element_type=jnp.float32)
    o = jnp.dot(o1, wp_ref[...], preferred_element_type=jnp.float32)
    o_ref[0] = o + bp_ref[...]


@jax.jit
def kernel(x, q, Wk, Wv, Wp, bp):
    B, N, C = x.shape
    CLS = q.shape[0]
    qp = jnp.pad(q, ((0, CLSP_ - CLS), (0, 0)))
    bp2 = bp.reshape(1, DIM_)
    grid = (B, CLSP_ // TQ_)
    out = pl.pallas_call(
        _body,
        grid=grid,
        in_specs=[
            pl.BlockSpec(memory_space=pl.ANY),
            pl.BlockSpec((TQ_, EMB_), lambda b, j: (j, 0)),
            pl.BlockSpec((DIM_, EMB_), lambda b, j: (0, 0)),
            pl.BlockSpec((DIM_, EMB_), lambda b, j: (0, 0)),
            pl.BlockSpec((EMB_, DIM_), lambda b, j: (0, 0)),
            pl.BlockSpec((1, DIM_), lambda b, j: (0, 0)),
        ],
        out_specs=pl.BlockSpec((1, TQ_, DIM_), lambda b, j: (b, j, 0)),
        out_shape=jax.ShapeDtypeStruct((B, CLSP_, DIM_), jnp.float32),
        scratch_shapes=[
            pltpu.VMEM((N_, DIM_), jnp.float32),
            pltpu.VMEM((N_, EMB_), jnp.float32),
            pltpu.VMEM((N_, EMB_), jnp.float32),
            pltpu.VMEM((TQ_, N_), jnp.float32),
            pltpu.VMEM((TQ_, N_), jnp.float32),
            pltpu.SemaphoreType.DMA,
        ],
    )(x, qp, Wk, Wv, Wp, bp2)
    return out[:, :CLS, :]
